# baseline (speedup 1.0000x reference)
import functools

import jax
import jax.numpy as jnp
from jax import lax
from jax.experimental import pallas as pl
from jax.experimental.pallas import tpu as pltpu
from jax.experimental.pallas import tpu_sc as plsc

_N_EMBED = 64
_BATCH = 16384
_N_SAMPLES = 5
_NC, _NS = 2, 16
_NW = _NC * _NS
_PERW = _BATCH // _NW
_NTASK = 2 + _N_SAMPLES
_CHUNK = 128
_NCH = _NTASK * _PERW // _CHUNK
_CPT = _PERW // _CHUNK


def _sg_body(tin2, tout2, idx_all, o_in, o_out, o_nz,
             vidx, vpair, ga, gb, sa, sb, gsem_a, gsem_b, ssem_a, ssem_b):
    wid = lax.axis_index("s") * _NC + lax.axis_index("c")
    gbufs = (ga, gb)
    gsems = (gsem_a, gsem_b)
    sbufs = (sa, sb)
    ssems = (ssem_a, ssem_b)

    tables = [tin2] + [tout2] * (_NTASK - 1)
    outs = [o_in, o_out] + [o_nz.at[n] for n in range(_N_SAMPLES)]

    for t in range(_NTASK):
        pltpu.sync_copy(idx_all.at[pl.ds(t * _BATCH + wid * _PERW, _PERW)],
                        vidx.at[pl.ds(t * _PERW, _PERW)])

    def halve(k, carry):
        off = pl.multiple_of(k * 16, 16)
        vpair[pl.ds(off, 16)] = jax.lax.shift_right_logical(
            vidx[pl.ds(off, 16)], 1)
        return carry
    lax.fori_loop(0, _NTASK * _PERW // 16, halve, 0)

    def fire_gather(i):
        t = i // _CPT
        return pltpu.async_copy(
            tables[t].at[vpair.at[pl.ds(i * _CHUNK, _CHUNK)]],
            gbufs[i % 2], gsems[i % 2])

    iota = lax.iota(jnp.int32, 16)

    def transpose_chunk(i):
        g = gbufs[i % 2]
        s = sbufs[(i // _CPT) % 2]
        jbase = (i % _CPT) * _CHUNK

        def group(l, carry):
            loff = pl.multiple_of(l * 16, 16)
            hvec = lax.bitwise_and(vidx[pl.ds(i * _CHUNK + loff, 16)], 1)
            rvec = loff + iota
            jvec = jbase + loff + iota
            cbase = hvec * _N_EMBED

            def dim(d, carry2):
                dvec = jnp.full((16,), 0, jnp.int32) + d
                vals = plsc.load_gather(g, [rvec, cbase + dvec])
                plsc.store_scatter(s, [dvec, jvec], vals)
                return carry2
            lax.fori_loop(0, _N_EMBED, dim, 0)
            return carry
        lax.fori_loop(0, _CHUNK // 16, group, 0)

    sd = [None] * _NTASK
    gd = [None] * _NCH
    gd[0] = fire_gather(0)
    gd[1] = fire_gather(1)
    for i in range(_NCH):
        t = i // _CPT
        if i % _CPT == 0 and t >= 2:
            sd[t - 2].wait()
        gd[i].wait()
        transpose_chunk(i)
        if i + 2 < _NCH:
            gd[i + 2] = fire_gather(i + 2)
        if i % _CPT == _CPT - 1:
            sd[t] = pltpu.async_copy(
                sbufs[t % 2],
                outs[t].at[:, pl.ds(wid * _PERW, _PERW)],
                ssems[t % 2])
    sd[_NTASK - 2].wait()
    sd[_NTASK - 1].wait()


@jax.jit
def _sg_gather(idx_all, tin2, tout2):
    mesh = plsc.VectorSubcoreMesh(core_axis_name="c", subcore_axis_name="s")
    f = functools.partial(
        pl.kernel,
        mesh=mesh,
        compiler_params=pltpu.CompilerParams(needs_layout_passes=False),
        out_type=(
            jax.ShapeDtypeStruct((_N_EMBED, _BATCH), jnp.float32),
            jax.ShapeDtypeStruct((_N_EMBED, _BATCH), jnp.float32),
            jax.ShapeDtypeStruct((_N_SAMPLES, _N_EMBED, _BATCH), jnp.float32),
        ),
        scratch_types=[
            pltpu.VMEM((_NTASK * _PERW,), jnp.int32),
            pltpu.VMEM((_NTASK * _PERW,), jnp.int32),
            pltpu.VMEM((_CHUNK, 2 * _N_EMBED), jnp.float32),
            pltpu.VMEM((_CHUNK, 2 * _N_EMBED), jnp.float32),
            pltpu.VMEM((_N_EMBED, _PERW), jnp.float32),
            pltpu.VMEM((_N_EMBED, _PERW), jnp.float32),
            pltpu.SemaphoreType.DMA,
            pltpu.SemaphoreType.DMA,
            pltpu.SemaphoreType.DMA,
            pltpu.SemaphoreType.DMA,
        ],
    )(_sg_body)
    return f(tin2, tout2, idx_all)


def kernel(input_words, output_words, noise_words, in_table, out_table):
    iw = input_words.astype(jnp.int32)
    ow = output_words.astype(jnp.int32)
    nzw = noise_words.astype(jnp.int32).reshape(_BATCH, _N_SAMPLES).T
    idx_all = jnp.concatenate([iw, ow, nzw.reshape(-1)])
    tin2 = in_table.reshape(500000, 2 * _N_EMBED)
    tout2 = out_table.reshape(500000, 2 * _N_EMBED)
    o_in, o_out, o_nz = _sg_gather(idx_all, tin2, tout2)
    return o_in.T, o_out.T, jnp.transpose(o_nz, (2, 0, 1))


# speedup vs baseline: 1.0431x; 1.0431x over previous
import functools

import jax
import jax.numpy as jnp
from jax import lax
from jax.experimental import pallas as pl
from jax.experimental.pallas import tpu as pltpu
from jax.experimental.pallas import tpu_sc as plsc

_N_EMBED = 64
_BATCH = 16384
_N_SAMPLES = 5
_NC, _NS = 2, 16
_NW = _NC * _NS
_CHUNK = 128
_NBUF = 3


def _pair_gather_body(nchunks, table2, idx3, out, vidx, b0, b1, b2,
                      g0, g1, g2, s0, s1, s2):
    wid = lax.axis_index("s") * _NC + lax.axis_index("c")
    bufs = (b0, b1, b2)
    gsems = (g0, g1, g2)
    ssems = (s0, s1, s2)
    base = wid * nchunks * _CHUNK

    pltpu.sync_copy(idx3.at[wid], vidx)

    def fire_gather(i):
        return pltpu.async_copy(table2.at[vidx.at[i]], bufs[i % _NBUF],
                                gsems[i % _NBUF])

    def fire_store(i):
        return pltpu.async_copy(bufs[i % _NBUF],
                                out.at[pl.ds(base + i * _CHUNK, _CHUNK)],
                                ssems[i % _NBUF])

    gd = [None] * nchunks
    sd = [None] * nchunks
    gd[0] = fire_gather(0)
    if nchunks > 1:
        gd[1] = fire_gather(1)
    for i in range(nchunks):
        gd[i].wait()
        sd[i] = fire_store(i)
        u = i + 2
        if u < nchunks:
            if u >= _NBUF:
                sd[u - _NBUF].wait()
            gd[u] = fire_gather(u)
    for i in range(max(0, nchunks - _NBUF), nchunks):
        sd[i].wait()


def _make_pair_gather(n_rows):
    nchunks = n_rows // (_NW * _CHUNK)
    mesh = plsc.VectorSubcoreMesh(core_axis_name="c", subcore_axis_name="s")
    return functools.partial(
        pl.kernel,
        mesh=mesh,
        out_type=jax.ShapeDtypeStruct((n_rows, 2 * _N_EMBED), jnp.float32),
        scratch_types=[
            pltpu.VMEM((nchunks, _CHUNK), jnp.int32),
            pltpu.VMEM((_CHUNK, 2 * _N_EMBED), jnp.float32),
            pltpu.VMEM((_CHUNK, 2 * _N_EMBED), jnp.float32),
            pltpu.VMEM((_CHUNK, 2 * _N_EMBED), jnp.float32),
            pltpu.SemaphoreType.DMA,
            pltpu.SemaphoreType.DMA,
            pltpu.SemaphoreType.DMA,
            pltpu.SemaphoreType.DMA,
            pltpu.SemaphoreType.DMA,
            pltpu.SemaphoreType.DMA,
        ],
    )(functools.partial(_pair_gather_body, nchunks))


@jax.jit
def _sg_forward(iw, ow, nz, in_table, out_table):
    tin2 = in_table.reshape(500000, 2 * _N_EMBED)
    tout2 = out_table.reshape(500000, 2 * _N_EMBED)

    idx_b = jnp.concatenate([ow, nz])
    pairs_a = _make_pair_gather(_BATCH)(
        tin2, (iw >> 1).reshape(_NW, -1, _CHUNK))
    pairs_b = _make_pair_gather((1 + _N_SAMPLES) * _BATCH)(
        tout2, (idx_b >> 1).reshape(_NW, -1, _CHUNK))

    def half_select(pairs, idx):
        h = (idx & 1).astype(jnp.bool_)
        return jnp.where(h[:, None], pairs[:, _N_EMBED:], pairs[:, :_N_EMBED])

    iv = half_select(pairs_a, iw)
    ov = half_select(pairs_b[:_BATCH], ow)
    nv = half_select(pairs_b[_BATCH:], nz)
    nv = jnp.transpose(nv.reshape(_N_SAMPLES, _BATCH, _N_EMBED), (1, 0, 2))
    return iv, ov, nv


def kernel(input_words, output_words, noise_words, in_table, out_table):
    iw = input_words.astype(jnp.int32)
    ow = output_words.astype(jnp.int32)
    nz = noise_words.astype(jnp.int32).reshape(_BATCH, _N_SAMPLES).T.reshape(-1)
    return _sg_forward(iw, ow, nz, in_table, out_table)


# speedup vs baseline: 1.0468x; 1.0036x over previous
import functools

import jax
import jax.numpy as jnp
from jax import lax
from jax.experimental import pallas as pl
from jax.experimental.pallas import tpu as pltpu
from jax.experimental.pallas import tpu_sc as plsc

_N_EMBED = 64
_BATCH = 16384
_N_SAMPLES = 5
_NC, _NS = 1, 16
_NW = _NC * _NS
_CHUNK = 128
_NBUF = 3


def _pair_gather_body(nchunks, table2, idx3, out, vidx, b0, b1, b2,
                      g0, g1, g2, s0, s1, s2):
    wid = lax.axis_index("s") * _NC + lax.axis_index("c")
    bufs = (b0, b1, b2)
    gsems = (g0, g1, g2)
    ssems = (s0, s1, s2)
    base = wid * nchunks * _CHUNK

    pltpu.sync_copy(idx3.at[wid], vidx)

    def fire_gather(i):
        return pltpu.async_copy(table2.at[vidx.at[i]], bufs[i % _NBUF],
                                gsems[i % _NBUF])

    def fire_store(i):
        return pltpu.async_copy(bufs[i % _NBUF],
                                out.at[pl.ds(base + i * _CHUNK, _CHUNK)],
                                ssems[i % _NBUF])

    gd = [None] * nchunks
    sd = [None] * nchunks
    gd[0] = fire_gather(0)
    if nchunks > 1:
        gd[1] = fire_gather(1)
    for i in range(nchunks):
        gd[i].wait()
        sd[i] = fire_store(i)
        u = i + 2
        if u < nchunks:
            if u >= _NBUF:
                sd[u - _NBUF].wait()
            gd[u] = fire_gather(u)
    for i in range(max(0, nchunks - _NBUF), nchunks):
        sd[i].wait()


def _make_pair_gather(n_rows):
    nchunks = n_rows // (_NW * _CHUNK)
    mesh = plsc.VectorSubcoreMesh(core_axis_name="c", subcore_axis_name="s",
                                  num_cores=_NC)
    return functools.partial(
        pl.kernel,
        mesh=mesh,
        out_type=jax.ShapeDtypeStruct((n_rows, 2 * _N_EMBED), jnp.float32),
        scratch_types=[
            pltpu.VMEM((nchunks, _CHUNK), jnp.int32),
            pltpu.VMEM((_CHUNK, 2 * _N_EMBED), jnp.float32),
            pltpu.VMEM((_CHUNK, 2 * _N_EMBED), jnp.float32),
            pltpu.VMEM((_CHUNK, 2 * _N_EMBED), jnp.float32),
            pltpu.SemaphoreType.DMA,
            pltpu.SemaphoreType.DMA,
            pltpu.SemaphoreType.DMA,
            pltpu.SemaphoreType.DMA,
            pltpu.SemaphoreType.DMA,
            pltpu.SemaphoreType.DMA,
        ],
    )(functools.partial(_pair_gather_body, nchunks))


@jax.jit
def _sg_forward(iw, ow, nz, in_table, out_table):
    tin2 = in_table.reshape(500000, 2 * _N_EMBED)
    tout2 = out_table.reshape(500000, 2 * _N_EMBED)

    idx_b = jnp.concatenate([ow, nz])
    pairs_a = _make_pair_gather(_BATCH)(
        tin2, (iw >> 1).reshape(_NW, -1, _CHUNK))
    pairs_b = _make_pair_gather((1 + _N_SAMPLES) * _BATCH)(
        tout2, (idx_b >> 1).reshape(_NW, -1, _CHUNK))

    def half_select(pairs, idx):
        h = (idx & 1).astype(jnp.bool_)
        return jnp.where(h[:, None], pairs[:, _N_EMBED:], pairs[:, :_N_EMBED])

    iv = half_select(pairs_a, iw)
    ov = half_select(pairs_b[:_BATCH], ow)
    nv = half_select(pairs_b[_BATCH:], nz)
    nv = jnp.transpose(nv.reshape(_N_SAMPLES, _BATCH, _N_EMBED), (1, 0, 2))
    return iv, ov, nv


def kernel(input_words, output_words, noise_words, in_table, out_table):
    iw = input_words.astype(jnp.int32)
    ow = output_words.astype(jnp.int32)
    nz = noise_words.astype(jnp.int32).reshape(_BATCH, _N_SAMPLES).T.reshape(-1)
    return _sg_forward(iw, ow, nz, in_table, out_table)


# speedup vs baseline: 1.0691x; 1.0213x over previous
import functools

import jax
import jax.numpy as jnp
from jax import lax
from jax.experimental import pallas as pl
from jax.experimental.pallas import tpu as pltpu
from jax.experimental.pallas import tpu_sc as plsc

_N_EMBED = 64
_BATCH = 16384
_N_SAMPLES = 5
_NC, _NS = 2, 16
_NW = _NC * _NS
_CHUNK = 128
_SUPER = 512
_NBUF = 3

_B_IN = _BATCH // _NW
_B_NZ = _BATCH * _N_SAMPLES // _NW
_C_IN = _B_IN // _CHUNK
_C_NZ = _B_NZ // _CHUNK
_S_IN = _B_IN // _SUPER
_S_NZ = _B_NZ // _SUPER
_NTASK = 2 * _S_IN + _S_NZ


def _sg_body(in_table, out_table, idx_in, idx_out, idx_nz,
             o_in, o_out, o_nz, v_in, v_out, v_nz,
             b0, b1, b2, g0, g1, g2, s0, s1, s2):
    wid = lax.axis_index("s") * _NC + lax.axis_index("c")
    bufs = (b0, b1, b2)
    gsems = (g0, g1, g2)
    ssems = (s0, s1, s2)

    pltpu.sync_copy(idx_in.at[wid], v_in)
    pltpu.sync_copy(idx_out.at[wid], v_out)
    pltpu.sync_copy(idx_nz.at[wid], v_nz)

    tasks = []
    for s in range(_S_IN):
        tasks.append((in_table, v_in, 4 * s, o_in, wid * _B_IN + s * _SUPER))
    for s in range(_S_IN):
        tasks.append((out_table, v_out, 4 * s, o_out, wid * _B_IN + s * _SUPER))
    for s in range(_S_NZ):
        tasks.append((out_table, v_nz, 4 * s, o_nz, wid * _B_NZ + s * _SUPER))

    def fire_gathers(t):
        table, vidx, crow, _, _ = tasks[t]
        b = t % _NBUF
        return [
            pltpu.async_copy(
                table.at[vidx.at[crow + k]],
                bufs[b].at[pl.ds(k * _CHUNK, _CHUNK)],
                gsems[b],
            )
            for k in range(4)
        ]

    def fire_store(t):
        _, _, _, out, orow = tasks[t]
        b = t % _NBUF
        return pltpu.async_copy(bufs[b], out.at[pl.ds(orow, _SUPER)], ssems[b])

    gd = [None] * _NTASK
    sd = [None] * _NTASK
    gd[0] = fire_gathers(0)
    gd[1] = fire_gathers(1)
    for t in range(_NTASK):
        for d in gd[t]:
            d.wait()
        sd[t] = fire_store(t)
        u = t + 2
        if u < _NTASK:
            if u >= _NBUF:
                sd[u - _NBUF].wait()
            gd[u] = fire_gathers(u)
    for t in range(_NTASK - _NBUF, _NTASK):
        sd[t].wait()


@jax.jit
def _sg_gather(iw, ow, nz, in_table, out_table):
    mesh = plsc.VectorSubcoreMesh(core_axis_name="c", subcore_axis_name="s")
    f = functools.partial(
        pl.kernel,
        mesh=mesh,
        compiler_params=pltpu.CompilerParams(use_tc_tiling_on_sc=False),
        out_type=(
            jax.ShapeDtypeStruct((_BATCH, _N_EMBED), jnp.float32),
            jax.ShapeDtypeStruct((_BATCH, _N_EMBED), jnp.float32),
            jax.ShapeDtypeStruct((_BATCH * _N_SAMPLES, _N_EMBED), jnp.float32),
        ),
        scratch_types=[
            pltpu.VMEM((_C_IN, _CHUNK), jnp.int32),
            pltpu.VMEM((_C_IN, _CHUNK), jnp.int32),
            pltpu.VMEM((_C_NZ, _CHUNK), jnp.int32),
            pltpu.VMEM((_SUPER, _N_EMBED), jnp.float32),
            pltpu.VMEM((_SUPER, _N_EMBED), jnp.float32),
            pltpu.VMEM((_SUPER, _N_EMBED), jnp.float32),
            pltpu.SemaphoreType.DMA,
            pltpu.SemaphoreType.DMA,
            pltpu.SemaphoreType.DMA,
            pltpu.SemaphoreType.DMA,
            pltpu.SemaphoreType.DMA,
            pltpu.SemaphoreType.DMA,
        ],
    )(_sg_body)
    return f(in_table, out_table, iw, ow, nz)


def kernel(input_words, output_words, noise_words, in_table, out_table):
    iw = input_words.astype(jnp.int32).reshape(_NW, _C_IN, _CHUNK)
    ow = output_words.astype(jnp.int32).reshape(_NW, _C_IN, _CHUNK)
    nz = noise_words.astype(jnp.int32).reshape(_NW, _C_NZ, _CHUNK)
    iv, ov, nv = _sg_gather(iw, ow, nz, in_table, out_table)
    return iv, ov, nv.reshape(_BATCH, _N_SAMPLES, _N_EMBED)
